# baseline (speedup 1.0000x reference)
import functools

import jax
import jax.numpy as jnp
from jax import lax
from jax.experimental import pallas as pl
from jax.experimental.pallas import tpu as pltpu
from jax.experimental.pallas import tpu_sc as plsc

B = 1024
D = 128
K = 10
POOL = 100000
CHUNK = 6144
NCHUNK = 17
PPAD = NCHUNK * CHUNK

_INF = float("inf")
_BIGI = 2**31 - 1

_NC = 2
_NS = 16
_NW = _NC * _NS
_BPW = (B * K) // _NW
_DT = 144


def _topk_body(x_ref, a2_ref, pool_ref, b2_ref, out_idx_ref, v_ref,
               rv_ref, ri_ref):
    j = pl.program_id(0)

    @pl.when(j == 0)
    def _init():
        rv_ref[...] = jnp.full_like(rv_ref, _INF)
        ri_ref[...] = jnp.full_like(ri_ref, _BIGI)

    mm = lax.dot_general(x_ref[...], pool_ref[...], (((1,), (1,)), ((), ())),
                         preferred_element_type=jnp.float32)
    d2 = a2_ref[...] + b2_ref[0] - 2.0 * mm
    v = jnp.sqrt(jnp.maximum(d2, 0.0))
    tau = rv_ref[:, K - 1:K]
    qual = v < tau
    v_ref[...] = jnp.where(qual, v, _INF)
    nmax = jnp.max(jnp.sum(qual.astype(jnp.int32), axis=1))

    colid = lax.broadcasted_iota(jnp.int32, (B, CHUNK), 1).astype(jnp.float32)
    c16 = lax.broadcasted_iota(jnp.int32, (B, 16), 1)
    base = j * CHUNK
    def _round(k, carry):
        vv = v_ref[...]
        m = jnp.min(vv, axis=1, keepdims=True)
        eq = vv == m
        ik = jnp.min(jnp.where(eq, colid, _INF), axis=1, keepdims=True)
        v_ref[...] = jnp.where(eq & (colid == ik), _INF, vv)
        gi = jnp.where(m < _INF, base + ik.astype(jnp.int32), _BIGI)

        rv = rv_ref[...]
        ri = ri_ref[...]
        less = (rv < m) | ((rv == m) & (ri < gi))
        pos = jnp.sum(less.astype(jnp.int32), axis=1, keepdims=True)
        rvs = jnp.concatenate([rv[:, :1], rv[:, :15]], axis=1)
        ris = jnp.concatenate([ri[:, :1], ri[:, :15]], axis=1)
        at = c16 == pos
        keep = c16 < pos
        rv_ref[...] = jnp.where(keep, rv, jnp.where(at, m, rvs))
        ri_ref[...] = jnp.where(keep, ri, jnp.where(at, gi, ris))
        return carry

    lax.fori_loop(0, jnp.minimum(nmax, K), _round, 0)

    @pl.when(j == NCHUNK - 1)
    def _out():
        out_idx_ref[...] = ri_ref[...]


_topk_call = pl.pallas_call(
    _topk_body,
    grid=(NCHUNK,),
    in_specs=[
        pl.BlockSpec((B, D), lambda j: (0, 0)),
        pl.BlockSpec((B, 1), lambda j: (0, 0)),
        pl.BlockSpec((CHUNK, D), lambda j: (j, 0)),
        pl.BlockSpec((1, 1, CHUNK), lambda j: (j, 0, 0)),
    ],
    out_specs=pl.BlockSpec((B, 16), lambda j: (0, 0)),
    out_shape=jax.ShapeDtypeStruct((B, 16), jnp.int32),
    scratch_shapes=[
        pltpu.VMEM((B, CHUNK), jnp.float32),
        pltpu.VMEM((B, 16), jnp.float32),
        pltpu.VMEM((B, 16), jnp.int32),
    ],
)


@functools.cache
def _sc_gather_fn():
    @functools.partial(
        pl.kernel,
        out_type=jax.ShapeDtypeStruct((B * K, _DT), jnp.float32),
        mesh=plsc.VectorSubcoreMesh(core_axis_name="c", subcore_axis_name="s",
                                    num_cores=_NC, num_subcores=_NS),
        scratch_types=[
            pltpu.VMEM((_BPW,), jnp.int32),
            pltpu.VMEM((_BPW, _DT), jnp.float32),
            pltpu.SemaphoreType.DMA,
        ],
        compiler_params=pltpu.CompilerParams(use_tc_tiling_on_sc=False),
    )
    def _sc_gather(table_hbm, idx_hbm, out_hbm, idx_v, rows_v, sem):
        wid = lax.axis_index("s") * _NC + lax.axis_index("c")
        base = wid * _BPW
        pltpu.sync_copy(idx_hbm.at[pl.ds(base, _BPW)], idx_v)
        pltpu.async_copy(table_hbm.at[idx_v], rows_v, sem).wait()
        pltpu.sync_copy(rows_v, out_hbm.at[pl.ds(base, _BPW)])

    return _sc_gather


def kernel(x, pool_x, pool_y):
    a2 = jnp.sum(x * x, axis=1, keepdims=True)
    b2 = jnp.sum(pool_x * pool_x, axis=1)
    b2p = jnp.pad(b2, (0, PPAD - POOL), constant_values=jnp.inf)
    b2p = b2p.reshape(NCHUNK, 1, CHUNK)
    poolp = jnp.pad(pool_x, ((0, PPAD - POOL), (0, 0)))

    idx16 = _topk_call(x, a2, poolp, b2p)
    idx = idx16[:, :K].reshape(B * K)

    table = jnp.concatenate(
        [pool_x, pool_y, jnp.zeros((POOL, _DT - D - 1), jnp.float32)], axis=1)
    rows = _sc_gather_fn()(table, idx)
    return rows[:, :D + 1].reshape(B, K, D + 1)


# speedup vs baseline: 1.2011x; 1.2011x over previous
import functools

import jax
import jax.numpy as jnp
from jax import lax
from jax.experimental import pallas as pl
from jax.experimental.pallas import tpu as pltpu
from jax.experimental.pallas import tpu_sc as plsc

B = 1024
D = 128
K = 10
POOL = 100000
CHUNK = 2048
NCHUNK = 49
PPAD = NCHUNK * CHUNK

_INF = float("inf")
_BIGI = 2**31 - 1

_NC = 2
_NS = 16
_NW = _NC * _NS
_BPW = (B * K) // _NW
_DT = 144


def _topk_body(x_ref, a2_ref, pool_ref, b2_ref, out_idx_ref, v_ref,
               rv_ref, ri_ref):
    j = pl.program_id(0)

    @pl.when(j == 0)
    def _init():
        rv_ref[...] = jnp.full_like(rv_ref, _INF)
        ri_ref[...] = jnp.full_like(ri_ref, _BIGI)

    mm = lax.dot_general(x_ref[...], pool_ref[...], (((1,), (1,)), ((), ())),
                         preferred_element_type=jnp.float32)
    d2 = a2_ref[...] + b2_ref[0] - 2.0 * mm
    v = jnp.sqrt(jnp.maximum(d2, 0.0))
    tau = rv_ref[:, K - 1:K]
    qual = v < tau
    v_ref[...] = jnp.where(qual, v, _INF)
    nmax = jnp.max(jnp.sum(qual.astype(jnp.int32), axis=1))

    colid = lax.broadcasted_iota(jnp.int32, (B, CHUNK), 1).astype(jnp.float32)
    c16 = lax.broadcasted_iota(jnp.int32, (B, 16), 1)
    base = j * CHUNK
    def _round(k, carry):
        vv = v_ref[...]
        m = jnp.min(vv, axis=1, keepdims=True)
        eq = vv == m
        ik = jnp.min(jnp.where(eq, colid, _INF), axis=1, keepdims=True)
        v_ref[...] = jnp.where(eq & (colid == ik), _INF, vv)
        gi = jnp.where(m < _INF, base + ik.astype(jnp.int32), _BIGI)

        rv = rv_ref[...]
        ri = ri_ref[...]
        less = (rv < m) | ((rv == m) & (ri < gi))
        pos = jnp.sum(less.astype(jnp.int32), axis=1, keepdims=True)
        rvs = jnp.concatenate([rv[:, :1], rv[:, :15]], axis=1)
        ris = jnp.concatenate([ri[:, :1], ri[:, :15]], axis=1)
        at = c16 == pos
        keep = c16 < pos
        rv_ref[...] = jnp.where(keep, rv, jnp.where(at, m, rvs))
        ri_ref[...] = jnp.where(keep, ri, jnp.where(at, gi, ris))
        return carry

    lax.fori_loop(0, jnp.minimum(nmax, K), _round, 0)

    @pl.when(j == NCHUNK - 1)
    def _out():
        out_idx_ref[...] = ri_ref[...]


_topk_call = pl.pallas_call(
    _topk_body,
    grid=(NCHUNK,),
    in_specs=[
        pl.BlockSpec((B, D), lambda j: (0, 0)),
        pl.BlockSpec((B, 1), lambda j: (0, 0)),
        pl.BlockSpec((CHUNK, D), lambda j: (j, 0)),
        pl.BlockSpec((1, 1, CHUNK), lambda j: (j, 0, 0)),
    ],
    out_specs=pl.BlockSpec((B, 16), lambda j: (0, 0)),
    out_shape=jax.ShapeDtypeStruct((B, 16), jnp.int32),
    scratch_shapes=[
        pltpu.VMEM((B, CHUNK), jnp.float32),
        pltpu.VMEM((B, 16), jnp.float32),
        pltpu.VMEM((B, 16), jnp.int32),
    ],
)


@functools.cache
def _sc_gather_fn():
    @functools.partial(
        pl.kernel,
        out_type=jax.ShapeDtypeStruct((B * K, _DT), jnp.float32),
        mesh=plsc.VectorSubcoreMesh(core_axis_name="c", subcore_axis_name="s",
                                    num_cores=_NC, num_subcores=_NS),
        scratch_types=[
            pltpu.VMEM((_BPW,), jnp.int32),
            pltpu.VMEM((_BPW, _DT), jnp.float32),
            pltpu.SemaphoreType.DMA,
        ],
        compiler_params=pltpu.CompilerParams(use_tc_tiling_on_sc=False),
    )
    def _sc_gather(table_hbm, idx_hbm, out_hbm, idx_v, rows_v, sem):
        wid = lax.axis_index("s") * _NC + lax.axis_index("c")
        base = wid * _BPW
        pltpu.sync_copy(idx_hbm.at[pl.ds(base, _BPW)], idx_v)
        pltpu.async_copy(table_hbm.at[idx_v], rows_v, sem).wait()
        pltpu.sync_copy(rows_v, out_hbm.at[pl.ds(base, _BPW)])

    return _sc_gather


def kernel(x, pool_x, pool_y):
    a2 = jnp.sum(x * x, axis=1, keepdims=True)
    b2 = jnp.sum(pool_x * pool_x, axis=1)
    b2p = jnp.pad(b2, (0, PPAD - POOL), constant_values=jnp.inf)
    b2p = b2p.reshape(NCHUNK, 1, CHUNK)
    poolp = jnp.pad(pool_x, ((0, PPAD - POOL), (0, 0)))

    idx16 = _topk_call(x, a2, poolp, b2p)
    idx = idx16[:, :K].reshape(B * K)

    table = jnp.concatenate(
        [pool_x, pool_y, jnp.zeros((POOL, _DT - D - 1), jnp.float32)], axis=1)
    rows = _sc_gather_fn()(table, idx)
    return rows[:, :D + 1].reshape(B, K, D + 1)


# speedup vs baseline: 1.2454x; 1.0369x over previous
import functools

import jax
import jax.numpy as jnp
from jax import lax
from jax.experimental import pallas as pl
from jax.experimental.pallas import tpu as pltpu
from jax.experimental.pallas import tpu_sc as plsc

B = 1024
D = 128
K = 10
POOL = 100000
CHUNK = 1024
NCHUNK = 98
PPAD = NCHUNK * CHUNK

_INF = float("inf")
_BIGI = 2**31 - 1

_NC = 2
_NS = 16
_NW = _NC * _NS
_BPW = (B * K) // _NW
_DT = 144


def _topk_body(x_ref, a2_ref, pool_ref, b2_ref, out_idx_ref, v_ref,
               rv_ref, ri_ref):
    j = pl.program_id(0)

    @pl.when(j == 0)
    def _init():
        rv_ref[...] = jnp.full_like(rv_ref, _INF)
        ri_ref[...] = jnp.full_like(ri_ref, _BIGI)

    mm = lax.dot_general(x_ref[...], pool_ref[...], (((1,), (1,)), ((), ())),
                         preferred_element_type=jnp.float32)
    d2 = a2_ref[...] + b2_ref[0] - 2.0 * mm
    v = jnp.sqrt(jnp.maximum(d2, 0.0))
    tau = rv_ref[:, K - 1:K]
    qual = v < tau
    v_ref[...] = jnp.where(qual, v, _INF)
    nmax = jnp.max(jnp.sum(qual.astype(jnp.int32), axis=1))

    colid = lax.broadcasted_iota(jnp.int32, (B, CHUNK), 1).astype(jnp.float32)
    c16 = lax.broadcasted_iota(jnp.int32, (B, 16), 1)
    base = j * CHUNK
    def _round(k, carry):
        vv = v_ref[...]
        m = jnp.min(vv, axis=1, keepdims=True)
        eq = vv == m
        ik = jnp.min(jnp.where(eq, colid, _INF), axis=1, keepdims=True)
        v_ref[...] = jnp.where(eq & (colid == ik), _INF, vv)
        gi = jnp.where(m < _INF, base + ik.astype(jnp.int32), _BIGI)

        rv = rv_ref[...]
        ri = ri_ref[...]
        less = (rv < m) | ((rv == m) & (ri < gi))
        pos = jnp.sum(less.astype(jnp.int32), axis=1, keepdims=True)
        rvs = jnp.concatenate([rv[:, :1], rv[:, :15]], axis=1)
        ris = jnp.concatenate([ri[:, :1], ri[:, :15]], axis=1)
        at = c16 == pos
        keep = c16 < pos
        rv_ref[...] = jnp.where(keep, rv, jnp.where(at, m, rvs))
        ri_ref[...] = jnp.where(keep, ri, jnp.where(at, gi, ris))
        return carry

    lax.fori_loop(0, jnp.minimum(nmax, K), _round, 0)

    @pl.when(j == NCHUNK - 1)
    def _out():
        out_idx_ref[...] = ri_ref[...]


_topk_call = pl.pallas_call(
    _topk_body,
    grid=(NCHUNK,),
    in_specs=[
        pl.BlockSpec((B, D), lambda j: (0, 0)),
        pl.BlockSpec((B, 1), lambda j: (0, 0)),
        pl.BlockSpec((CHUNK, D), lambda j: (j, 0)),
        pl.BlockSpec((1, 1, CHUNK), lambda j: (j, 0, 0)),
    ],
    out_specs=pl.BlockSpec((B, 16), lambda j: (0, 0)),
    out_shape=jax.ShapeDtypeStruct((B, 16), jnp.int32),
    scratch_shapes=[
        pltpu.VMEM((B, CHUNK), jnp.float32),
        pltpu.VMEM((B, 16), jnp.float32),
        pltpu.VMEM((B, 16), jnp.int32),
    ],
)


@functools.cache
def _sc_gather_fn():
    @functools.partial(
        pl.kernel,
        out_type=jax.ShapeDtypeStruct((B * K, _DT), jnp.float32),
        mesh=plsc.VectorSubcoreMesh(core_axis_name="c", subcore_axis_name="s",
                                    num_cores=_NC, num_subcores=_NS),
        scratch_types=[
            pltpu.VMEM((_BPW,), jnp.int32),
            pltpu.VMEM((_BPW, _DT), jnp.float32),
            pltpu.SemaphoreType.DMA,
        ],
        compiler_params=pltpu.CompilerParams(use_tc_tiling_on_sc=False),
    )
    def _sc_gather(table_hbm, idx_hbm, out_hbm, idx_v, rows_v, sem):
        wid = lax.axis_index("s") * _NC + lax.axis_index("c")
        base = wid * _BPW
        pltpu.sync_copy(idx_hbm.at[pl.ds(base, _BPW)], idx_v)
        pltpu.async_copy(table_hbm.at[idx_v], rows_v, sem).wait()
        pltpu.sync_copy(rows_v, out_hbm.at[pl.ds(base, _BPW)])

    return _sc_gather


def kernel(x, pool_x, pool_y):
    a2 = jnp.sum(x * x, axis=1, keepdims=True)
    b2 = jnp.sum(pool_x * pool_x, axis=1)
    b2p = jnp.pad(b2, (0, PPAD - POOL), constant_values=jnp.inf)
    b2p = b2p.reshape(NCHUNK, 1, CHUNK)
    poolp = jnp.pad(pool_x, ((0, PPAD - POOL), (0, 0)))

    idx16 = _topk_call(x, a2, poolp, b2p)
    idx = idx16[:, :K].reshape(B * K)

    table = jnp.concatenate(
        [pool_x, pool_y, jnp.zeros((POOL, _DT - D - 1), jnp.float32)], axis=1)
    rows = _sc_gather_fn()(table, idx)
    return rows[:, :D + 1].reshape(B, K, D + 1)
